# baseline (speedup 1.0000x reference)
import functools

import jax
import jax.numpy as jnp
from jax import lax
from jax.experimental import pallas as pl
from jax.experimental.pallas import tpu as pltpu
from jax.experimental.pallas import tpu_sc as plsc

CLASSES = 19
PADC = 32
DECAY = 0.99
TEMPERATURE = 0.5
EPS = 1e-07
NEG_BIG = -1e30

NC = 2
NS = 16
NW = NC * NS
L = 16

N = 16 * 512 * 512
Q = N // 4
PER_W = Q // NW
CHUNK_J = 8192
NCHUNK = PER_W // CHUNK_J

_mesh = plsc.VectorSubcoreMesh(core_axis_name="c", subcore_axis_name="s")
_params = pltpu.CompilerParams(needs_layout_passes=False)
_params_tiled = pltpu.CompilerParams(
    needs_layout_passes=False, use_tc_tiling_on_sc=True)

ROWS_PER_W = 256
ROWS_PER_CH = 64
NCHUNK_H = ROWS_PER_W // ROWS_PER_CH


def _worker_id():
    return lax.axis_index("s") * NC + lax.axis_index("c")


@functools.partial(
    pl.kernel,
    mesh=_mesh,
    compiler_params=_params_tiled,
    out_type=jax.ShapeDtypeStruct((NW, PADC), jnp.int32),
    scratch_types=[
        pltpu.VMEM((2, ROWS_PER_CH, 512), jnp.int32),
        pltpu.VMEM((L * PADC,), jnp.int32),
        pltpu.VMEM((PADC,), jnp.int32),
        pltpu.SemaphoreType.DMA,
        pltpu.SemaphoreType.DMA,
    ],
)
def _hist_kernel(label_hbm, part_hbm, lbl_v, tab_v, cnt_v, sem0, sem1):
    wid = _worker_id()
    img = wid // 2
    row0 = (wid % 2) * ROWS_PER_W
    sems = (sem0, sem1)

    cps = [None, None]
    cps[0] = pltpu.async_copy(
        label_hbm.at[img, pl.ds(row0, ROWS_PER_CH), :], lbl_v.at[0], sems[0])

    zero16 = jnp.zeros((L,), jnp.int32)
    ones16 = jnp.ones((L,), jnp.int32)
    lane_off = lax.iota(jnp.int32, L) * PADC

    for r in range(PADC):
        tab_v[pl.ds(r * L, L)] = zero16

    for ch in range(NCHUNK_H):
        b = ch & 1
        if ch + 1 < NCHUNK_H:
            cps[1 - b] = pltpu.async_copy(
                label_hbm.at[img, pl.ds(row0 + (ch + 1) * ROWS_PER_CH,
                                        ROWS_PER_CH), :],
                lbl_v.at[1 - b], sems[1 - b])
        cps[b].wait()

        @plsc.parallel_loop(0, ROWS_PER_CH * 512, step=L, unroll=8)
        def _(i):
            v = lbl_v[b, i >> 9, pl.ds(i & 511, L)]
            plsc.addupdate_scatter(tab_v, [lane_off + v], ones16)

    c0 = jnp.zeros((L,), jnp.int32)
    c1 = jnp.zeros((L,), jnp.int32)
    for r in range(L):
        c0 = c0 + tab_v[pl.ds(r * PADC, L)]
        c1 = c1 + tab_v[pl.ds(r * PADC + L, L)]
    cnt_v[pl.ds(0, L)] = c0
    cnt_v[pl.ds(L, L)] = c1
    pltpu.sync_copy(cnt_v, part_hbm.at[wid])


@functools.partial(
    pl.kernel,
    mesh=_mesh,
    compiler_params=_params,
    out_type=(
        jax.ShapeDtypeStruct((N,), jnp.float32),
        jax.ShapeDtypeStruct((PADC,), jnp.float32),
    ),
    scratch_types=[
        pltpu.VMEM((NW, PADC), jnp.int32),
        pltpu.VMEM((PADC,), jnp.float32),
        pltpu.VMEM((PADC,), jnp.float32),
        pltpu.VMEM((2 * CHUNK_J,), jnp.int32),
        pltpu.VMEM((2 * 4 * CHUNK_J,), jnp.float32),
        pltpu.SemaphoreType.DMA,
        pltpu.SemaphoreType.DMA,
        pltpu.SemaphoreType.DMA,
        pltpu.SemaphoreType.DMA,
    ],
)
def _gather_kernel(words_hbm, part_hbm, freq_hbm, weight_hbm, fnew_hbm,
                   part_v, freq_v, prob_v, lbl_v, w_v,
                   si0, si1, so0, so1):
    wid = _worker_id()
    wbase = wid * PER_W
    sin = (si0, si1)
    sout = (so0, so1)

    in_cp = [None, None]
    in_cp[0] = pltpu.async_copy(
        words_hbm.at[pl.ds(wbase, CHUNK_J)],
        lbl_v.at[pl.ds(0, CHUNK_J)], sin[0])

    pltpu.sync_copy(part_hbm, part_v)
    pltpu.sync_copy(freq_hbm, freq_v)

    c0 = jnp.zeros((L,), jnp.int32)
    c1 = jnp.zeros((L,), jnp.int32)
    for r in range(NW):
        c0 = c0 + part_v[r, pl.ds(0, L)]
        c1 = c1 + part_v[r, pl.ds(L, L)]

    inv_total = 1.0 / (float(N) + EPS)
    cf0 = c0.astype(jnp.float32) * inv_total
    cf1 = c1.astype(jnp.float32) * inv_total
    fn0 = (1.0 - DECAY) * cf0 + DECAY * freq_v[pl.ds(0, L)]
    fn1 = (1.0 - DECAY) * cf1 + DECAY * freq_v[pl.ds(L, L)]

    valid1 = lax.iota(jnp.int32, L) < (CLASSES - L)
    x0 = (1.0 - fn0) / TEMPERATURE
    x1 = (1.0 - fn1) / TEMPERATURE
    m = jnp.maximum(jnp.max(x0), jnp.max(jnp.where(valid1, x1, NEG_BIG)))
    e0 = jnp.exp(x0 - m)
    e1 = jnp.where(valid1, jnp.exp(x1 - m), 0.0)
    s = jnp.sum(e0) + jnp.sum(e1)
    p0 = e0 / s
    p1 = e1 / s
    pmax = jnp.maximum(jnp.max(p0), jnp.max(p1))
    pn0 = p0 / (pmax + EPS)
    pn1 = p1 / (pmax + EPS)
    prob_v[pl.ds(0, L)] = pn0
    prob_v[pl.ds(L, L)] = pn1

    @pl.when(wid == 0)
    def _():
        freq_v[pl.ds(0, L)] = fn0
        freq_v[pl.ds(L, L)] = fn1
        pltpu.sync_copy(freq_v, fnew_hbm)

    mask8 = jnp.full((L,), 0xFF, jnp.int32)

    out_cp = [[None] * 4, [None] * 4]
    for ch in range(NCHUNK):
        b = ch & 1
        if ch + 1 < NCHUNK:
            in_cp[1 - b] = pltpu.async_copy(
                words_hbm.at[pl.ds(wbase + (ch + 1) * CHUNK_J, CHUNK_J)],
                lbl_v.at[pl.ds((1 - b) * CHUNK_J, CHUNK_J)], sin[1 - b])
        in_cp[b].wait()
        for k in range(4):
            if out_cp[b][k] is not None:
                out_cp[b][k].wait()
                out_cp[b][k] = None

        @plsc.parallel_loop(0, CHUNK_J, step=L, unroll=8)
        def _(i):
            w = lbl_v[pl.ds(b * CHUNK_J + i, L)]
            for k in range(4):
                idx = jnp.bitwise_and(
                    lax.shift_right_logical(w, jnp.int32(8 * k)), mask8)
                w_v[pl.ds((b * 4 + k) * CHUNK_J + i, L)] = (
                    plsc.load_gather(prob_v, [idx]))

        for k in range(4):
            out_cp[b][k] = pltpu.async_copy(
                w_v.at[pl.ds((b * 4 + k) * CHUNK_J, CHUNK_J)],
                weight_hbm.at[pl.ds(k * Q + wbase + ch * CHUNK_J, CHUNK_J)],
                sout[b])

    for b in range(2):
        for k in range(4):
            if out_cp[b][k] is not None:
                out_cp[b][k].wait()


def kernel(label, freq):
    l4 = jnp.reshape(label, (4, 4, 512, 512)).astype(jnp.int32)
    words3 = l4[0] | (l4[1] << 8) | (l4[2] << 16) | (l4[3] << 24)
    words3 = lax.optimization_barrier(words3)
    words = jnp.reshape(words3, (Q,))
    freq_pad = jnp.zeros((PADC,), jnp.float32).at[:CLASSES].set(
        freq.astype(jnp.float32))
    partials = _hist_kernel(label.astype(jnp.int32))
    weight, fnew_pad = _gather_kernel(words, partials, freq_pad)
    return weight, fnew_pad[:CLASSES]


# speedup vs baseline: 1.0566x; 1.0566x over previous
import functools

import jax
import jax.numpy as jnp
from jax import lax
from jax.experimental import pallas as pl
from jax.experimental.pallas import tpu as pltpu
from jax.experimental.pallas import tpu_sc as plsc

CLASSES = 19
PADC = 32
DECAY = 0.99
TEMPERATURE = 0.5
EPS = 1e-07
NEG_BIG = -1e30

NC = 2
NS = 16
NW = NC * NS
L = 16

N = 16 * 512 * 512
Q = N // 4
PER_W = Q // NW
CHUNK_J = 8192
NCHUNK = PER_W // CHUNK_J

_mesh = plsc.VectorSubcoreMesh(core_axis_name="c", subcore_axis_name="s")
_params = pltpu.CompilerParams(needs_layout_passes=False)
_params_tiled = pltpu.CompilerParams(
    needs_layout_passes=False, use_tc_tiling_on_sc=True)

ROWS_PER_W = 256
ROWS_PER_CH = 32
NCHUNK_H = ROWS_PER_W // ROWS_PER_CH


def _worker_id():
    return lax.axis_index("s") * NC + lax.axis_index("c")


@functools.partial(
    pl.kernel,
    mesh=_mesh,
    compiler_params=_params_tiled,
    out_type=jax.ShapeDtypeStruct((NW, PADC), jnp.int32),
    scratch_types=[
        pltpu.VMEM((2, ROWS_PER_CH, 512), jnp.int32),
        pltpu.VMEM((L * PADC * PADC,), jnp.int32),
        pltpu.VMEM((PADC * PADC,), jnp.int32),
        pltpu.VMEM((PADC,), jnp.int32),
        pltpu.SemaphoreType.DMA,
        pltpu.SemaphoreType.DMA,
    ],
)
def _hist_kernel(label_hbm, part_hbm, lbl_v, tab_v, tp_v, cnt_v, sem0, sem1):
    wid = _worker_id()
    img = wid // 2
    row0 = (wid % 2) * ROWS_PER_W
    sems = (sem0, sem1)

    cps = [None, None]
    cps[0] = pltpu.async_copy(
        label_hbm.at[img, pl.ds(row0, ROWS_PER_CH), :], lbl_v.at[0], sems[0])

    zero16 = jnp.zeros((L,), jnp.int32)
    ones16 = jnp.ones((L,), jnp.int32)
    lane_off = lax.iota(jnp.int32, L) * (PADC * PADC)

    @plsc.parallel_loop(0, L * PADC * PADC, step=L, unroll=8)
    def _(i):
        tab_v[pl.ds(i, L)] = zero16

    for ch in range(NCHUNK_H):
        b = ch & 1
        if ch + 1 < NCHUNK_H:
            cps[1 - b] = pltpu.async_copy(
                label_hbm.at[img, pl.ds(row0 + (ch + 1) * ROWS_PER_CH,
                                        ROWS_PER_CH), :],
                lbl_v.at[1 - b], sems[1 - b])
        cps[b].wait()

        @plsc.parallel_loop(0, ROWS_PER_CH * 512, step=2 * L, unroll=4)
        def _(i):
            r = i >> 9
            c = i & 511
            va = lbl_v[b, r, pl.ds(c, L)]
            vb = lbl_v[b, r, pl.ds(c + L, L)]
            idx = lane_off + (va << 5) + vb
            plsc.addupdate_scatter(tab_v, [idx], ones16)

    @plsc.parallel_loop(0, PADC * PADC, step=L, unroll=2)
    def _(p):
        acc = tab_v[pl.ds(p, L)]
        for l in range(1, L):
            acc = acc + tab_v[pl.ds(l * PADC * PADC + p, L)]
        tp_v[pl.ds(p, L)] = acc

    b0 = jnp.zeros((L,), jnp.int32)
    b1 = jnp.zeros((L,), jnp.int32)
    a0 = jnp.zeros((L,), jnp.int32)
    a1 = jnp.zeros((L,), jnp.int32)
    lane_iota = lax.iota(jnp.int32, L)
    for a in range(PADC):
        ra0 = tp_v[pl.ds(a * PADC, L)]
        ra1 = tp_v[pl.ds(a * PADC + L, L)]
        b0 = b0 + ra0
        b1 = b1 + ra1
        s = jnp.sum(ra0) + jnp.sum(ra1)
        if a < L:
            a0 = jnp.where(lane_iota == a, a0 + s, a0)
        else:
            a1 = jnp.where(lane_iota == (a - L), a1 + s, a1)
    cnt_v[pl.ds(0, L)] = a0 + b0
    cnt_v[pl.ds(L, L)] = a1 + b1
    pltpu.sync_copy(cnt_v, part_hbm.at[wid])


@functools.partial(
    pl.kernel,
    mesh=_mesh,
    compiler_params=_params,
    out_type=(
        jax.ShapeDtypeStruct((N,), jnp.float32),
        jax.ShapeDtypeStruct((PADC,), jnp.float32),
    ),
    scratch_types=[
        pltpu.VMEM((NW, PADC), jnp.int32),
        pltpu.VMEM((PADC,), jnp.float32),
        pltpu.VMEM((PADC,), jnp.float32),
        pltpu.VMEM((2 * CHUNK_J,), jnp.int32),
        pltpu.VMEM((2 * 4 * CHUNK_J,), jnp.float32),
        pltpu.SemaphoreType.DMA,
        pltpu.SemaphoreType.DMA,
        pltpu.SemaphoreType.DMA,
        pltpu.SemaphoreType.DMA,
    ],
)
def _gather_kernel(words_hbm, part_hbm, freq_hbm, weight_hbm, fnew_hbm,
                   part_v, freq_v, prob_v, lbl_v, w_v,
                   si0, si1, so0, so1):
    wid = _worker_id()
    wbase = wid * PER_W
    sin = (si0, si1)
    sout = (so0, so1)

    in_cp = [None, None]
    in_cp[0] = pltpu.async_copy(
        words_hbm.at[pl.ds(wbase, CHUNK_J)],
        lbl_v.at[pl.ds(0, CHUNK_J)], sin[0])

    pltpu.sync_copy(part_hbm, part_v)
    pltpu.sync_copy(freq_hbm, freq_v)

    c0 = jnp.zeros((L,), jnp.int32)
    c1 = jnp.zeros((L,), jnp.int32)
    for r in range(NW):
        c0 = c0 + part_v[r, pl.ds(0, L)]
        c1 = c1 + part_v[r, pl.ds(L, L)]

    inv_total = 1.0 / (float(N) + EPS)
    cf0 = c0.astype(jnp.float32) * inv_total
    cf1 = c1.astype(jnp.float32) * inv_total
    fn0 = (1.0 - DECAY) * cf0 + DECAY * freq_v[pl.ds(0, L)]
    fn1 = (1.0 - DECAY) * cf1 + DECAY * freq_v[pl.ds(L, L)]

    valid1 = lax.iota(jnp.int32, L) < (CLASSES - L)
    x0 = (1.0 - fn0) / TEMPERATURE
    x1 = (1.0 - fn1) / TEMPERATURE
    m = jnp.maximum(jnp.max(x0), jnp.max(jnp.where(valid1, x1, NEG_BIG)))
    e0 = jnp.exp(x0 - m)
    e1 = jnp.where(valid1, jnp.exp(x1 - m), 0.0)
    s = jnp.sum(e0) + jnp.sum(e1)
    p0 = e0 / s
    p1 = e1 / s
    pmax = jnp.maximum(jnp.max(p0), jnp.max(p1))
    pn0 = p0 / (pmax + EPS)
    pn1 = p1 / (pmax + EPS)
    prob_v[pl.ds(0, L)] = pn0
    prob_v[pl.ds(L, L)] = pn1

    @pl.when(wid == 0)
    def _():
        freq_v[pl.ds(0, L)] = fn0
        freq_v[pl.ds(L, L)] = fn1
        pltpu.sync_copy(freq_v, fnew_hbm)

    mask8 = jnp.full((L,), 0xFF, jnp.int32)

    out_cp = [[None] * 4, [None] * 4]
    for ch in range(NCHUNK):
        b = ch & 1
        if ch + 1 < NCHUNK:
            in_cp[1 - b] = pltpu.async_copy(
                words_hbm.at[pl.ds(wbase + (ch + 1) * CHUNK_J, CHUNK_J)],
                lbl_v.at[pl.ds((1 - b) * CHUNK_J, CHUNK_J)], sin[1 - b])
        in_cp[b].wait()
        for k in range(4):
            if out_cp[b][k] is not None:
                out_cp[b][k].wait()
                out_cp[b][k] = None

        @plsc.parallel_loop(0, CHUNK_J, step=L, unroll=8)
        def _(i):
            w = lbl_v[pl.ds(b * CHUNK_J + i, L)]
            for k in range(4):
                idx = jnp.bitwise_and(
                    lax.shift_right_logical(w, jnp.int32(8 * k)), mask8)
                w_v[pl.ds((b * 4 + k) * CHUNK_J + i, L)] = (
                    plsc.load_gather(prob_v, [idx]))

        for k in range(4):
            out_cp[b][k] = pltpu.async_copy(
                w_v.at[pl.ds((b * 4 + k) * CHUNK_J, CHUNK_J)],
                weight_hbm.at[pl.ds(k * Q + wbase + ch * CHUNK_J, CHUNK_J)],
                sout[b])

    for b in range(2):
        for k in range(4):
            if out_cp[b][k] is not None:
                out_cp[b][k].wait()


def kernel(label, freq):
    l4 = jnp.reshape(label, (4, 4, 512, 512)).astype(jnp.int32)
    words3 = l4[0] | (l4[1] << 8) | (l4[2] << 16) | (l4[3] << 24)
    words3 = lax.optimization_barrier(words3)
    words = jnp.reshape(words3, (Q,))
    freq_pad = jnp.zeros((PADC,), jnp.float32).at[:CLASSES].set(
        freq.astype(jnp.float32))
    partials = _hist_kernel(label.astype(jnp.int32))
    weight, fnew_pad = _gather_kernel(words, partials, freq_pad)
    return weight, fnew_pad[:CLASSES]
